# baseline (speedup 1.0000x reference)
import functools

import jax
import jax.numpy as jnp
from jax import lax
from jax.experimental import pallas as pl
from jax.experimental.pallas import tpu as pltpu
from jax.experimental.pallas import tpu_sc as plsc

NUM_CORES = 2
NUM_SUBCORES = 16
NUM_WORKERS = NUM_CORES * NUM_SUBCORES
CHUNK = 2048


def _mesh():
    return plsc.VectorSubcoreMesh(
        core_axis_name="c",
        subcore_axis_name="s",
        num_cores=NUM_CORES,
        num_subcores=NUM_SUBCORES,
    )


@functools.partial(jax.jit, static_argnames=("B", "D"))
def _gather_flat(idx_flat, W, B, D):
    per_w = B // NUM_WORKERS
    n_chunks = per_w // CHUNK
    assert n_chunks % 2 == 0 and n_chunks >= 4
    n_pairs = (n_chunks - 2) // 2

    @functools.partial(
        pl.kernel,
        mesh=_mesh(),
        compiler_params=pltpu.CompilerParams(use_tc_tiling_on_sc=False),
        out_type=jax.ShapeDtypeStruct((B, D), jnp.float32),
        scratch_types=[
            pltpu.VMEM((2, CHUNK), jnp.int32),
            pltpu.VMEM((2, CHUNK, D), jnp.float32),
            pltpu.SemaphoreType.DMA,
            pltpu.SemaphoreType.DMA,
        ],
    )
    def k(idx_hbm, table_hbm, out_hbm, idx_v, rows_v, gsem, osem):
        wid = lax.axis_index("s") * NUM_CORES + lax.axis_index("c")
        base = wid * per_w

        def load_idx(g, b):
            pltpu.sync_copy(idx_hbm.at[pl.ds(base + g * CHUNK, CHUNK)],
                            idx_v.at[b])

        def fire_gather(b):
            pltpu.async_copy(table_hbm.at[idx_v.at[b]], rows_v.at[b], gsem)

        def wait_gather(b):
            pltpu.make_async_copy(table_hbm.at[idx_v.at[b]], rows_v.at[b],
                                  gsem).wait()

        def fire_store(g, b):
            pltpu.async_copy(rows_v.at[b],
                             out_hbm.at[pl.ds(base + g * CHUNK, CHUNK)], osem)

        def wait_store(g, b):
            pltpu.make_async_copy(rows_v.at[b],
                                  out_hbm.at[pl.ds(base + g * CHUNK, CHUNK)],
                                  osem).wait()

        load_idx(0, 0)
        fire_gather(0)
        load_idx(1, 1)
        wait_gather(0)
        fire_gather(1)
        fire_store(0, 0)

        def step(g, b):
            load_idx(g + 1, b ^ 1)
            wait_gather(b)
            wait_store(g - 1, b ^ 1)
            fire_gather(b ^ 1)
            fire_store(g, b)

        def body(p, carry):
            step(2 * p + 1, 1)
            step(2 * p + 2, 0)
            return carry

        lax.fori_loop(0, n_pairs, body, 0)

        g_last = n_chunks - 1
        wait_gather(1)
        wait_store(g_last - 1, 0)
        fire_store(g_last, 1)
        wait_store(g_last, 1)

    return k(idx_flat, W)


WCV_FULL = 1000000 // 128
WCV_REM = 1000000 - WCV_FULL * 128


@jax.jit
def _wconv(Wt3, tail):
    V = Wt3.shape[2]

    @functools.partial(
        pl.kernel,
        mesh=_mesh(),
        compiler_params=pltpu.CompilerParams(
            use_tc_tiling_on_sc=True, needs_layout_passes=False),
        out_type=jax.ShapeDtypeStruct((V * 16,), jnp.float32),
        scratch_types=[
            pltpu.VMEM((2, 2, 1, 8, 128), jnp.float32),
            pltpu.VMEM((2, 2048), jnp.float32),
            pltpu.SemaphoreType.DMA,
            pltpu.SemaphoreType.DMA,
        ],
    )
    def k(wt_hbm, tail_hbm, out_hbm, wbuf, obuf, lsem, osem):
        wid = lax.axis_index("s") * NUM_CORES + lax.axis_index("c")
        extra = jnp.minimum(wid, WCV_FULL % NUM_WORKERS)
        nblk = WCV_FULL // NUM_WORKERS + jnp.where(
            wid < WCV_FULL % NUM_WORKERS, 1, 0)
        kstart = wid * (WCV_FULL // NUM_WORKERS) + extra
        d_hi = lax.iota(jnp.int32, 16) // 8
        d_lo = lax.iota(jnp.int32, 16) % 8
        zeros16 = jnp.full((16,), 0, jnp.int32)

        def fire_load(blk, b):
            for tr in range(2):
                pltpu.async_copy(
                    wt_hbm.at[pl.ds(tr, 1), :,
                              pl.ds((kstart + blk) * 128, 128)],
                    wbuf.at[b, tr], lsem)

        def wait_load(blk, b):
            for tr in range(2):
                pltpu.make_async_copy(
                    wt_hbm.at[pl.ds(tr, 1), :,
                              pl.ds((kstart + blk) * 128, 128)],
                    wbuf.at[b, tr], lsem).wait()

        def transpose(b):
            b_vec = jnp.full((16,), b, jnp.int32)
            for r in range(16):
                for c16 in range(8):
                    col = jnp.full((16,), r * 8 + c16, jnp.int32)
                    v = plsc.load_gather(
                        wbuf, [b_vec, d_hi, zeros16, d_lo, col])
                    plsc.store_scatter(
                        obuf,
                        [b_vec,
                         lax.iota(jnp.int32, 16) + (r * 128 + c16 * 16)], v)

        def fire_store(blk, b):
            pltpu.async_copy(
                obuf.at[b], out_hbm.at[pl.ds((kstart + blk) * 2048, 2048)],
                osem)

        def wait_store(blk, b):
            pltpu.make_async_copy(
                obuf.at[b], out_hbm.at[pl.ds((kstart + blk) * 2048, 2048)],
                osem).wait()

        fire_load(0, 0)
        fire_load(1, 1)
        wait_load(0, 0)
        transpose(0)
        fire_store(0, 0)

        def body(blk, carry):
            b = blk % 2
            fire_load(blk + 1, b ^ 1)
            wait_load(blk, b)
            wait_store(blk - 1, b ^ 1)
            transpose(b)
            fire_store(blk, b)
            return carry

        lax.fori_loop(1, nblk - 1, body, 0)

        b_last = (nblk - 1) % 2
        wait_load(nblk - 1, b_last)
        wait_store(nblk - 2, b_last ^ 1)
        transpose(b_last)
        fire_store(nblk - 1, b_last)
        wait_store(nblk - 1, b_last)

        @pl.when(wid == NUM_WORKERS - 1)
        def _tail():
            pltpu.sync_copy(tail_hbm, obuf.at[0, pl.ds(0, WCV_REM * 16)])
            pltpu.sync_copy(obuf.at[0, pl.ds(0, WCV_REM * 16)],
                            out_hbm.at[pl.ds(WCV_FULL * 2048, WCV_REM * 16)])

    return k(Wt3, tail)


UNITS = 1600
UNIT_ROWS = 2048
UNIT_ELEMS = UNIT_ROWS * 16


@functools.partial(jax.jit, static_argnames=("HIST", "BATCH"))
def _relayout(flat, HIST, BATCH):
    per_w = UNITS // NUM_WORKERS

    @functools.partial(
        pl.kernel,
        mesh=_mesh(),
        compiler_params=pltpu.CompilerParams(
            use_tc_tiling_on_sc=True, needs_layout_passes=False),
        out_type=jax.ShapeDtypeStruct((HIST, 16, BATCH), jnp.float32),
        scratch_types=[
            pltpu.VMEM((UNIT_ELEMS,), jnp.float32),
            pltpu.VMEM((UNIT_ELEMS,), jnp.float32),
            pltpu.VMEM((8, 2, 1, 8, 257), jnp.float32),
            pltpu.SemaphoreType.DMA,
            pltpu.SemaphoreType.DMA,
        ],
    )
    def k(in_hbm, out_hbm, buf0, buf1, tbuf, lsem, osem):
        wid = lax.axis_index("s") * NUM_CORES + lax.axis_index("c")
        ubase = wid * per_w
        iota = lax.iota(jnp.int32, 16)
        tr_v = iota // 8
        dd_v = iota % 8
        zero_v = jnp.full((16,), 0, jnp.int32)

        def fire_load(u, bufref):
            pltpu.async_copy(
                in_hbm.at[pl.ds((ubase + u) * UNIT_ELEMS, UNIT_ELEMS)],
                bufref, lsem)

        def wait_load(u, bufref):
            pltpu.make_async_copy(
                in_hbm.at[pl.ds((ubase + u) * UNIT_ELEMS, UNIT_ELEMS)],
                bufref, lsem).wait()

        def transpose(bufref):
            def p_body(p, carry):
                jr = p // 16
                icl = (p % 16) // 8
                ii16 = p % 8
                r0 = (icl * 8 + jr) * 128 + ii16 * 16
                col = icl * 128 + ii16 * 16
                jr_c = jnp.full((16,), jr, jnp.int32)
                for kk in range(16):
                    v = bufref[pl.ds((r0 + kk) * 16, 16)]
                    plsc.store_scatter(
                        tbuf,
                        [jr_c, tr_v, zero_v, dd_v,
                         jnp.full((16,), col + kk, jnp.int32)], v)
                return carry
            lax.fori_loop(0, 128, p_body, 0)

        def out_slice(u, jr, tr):
            ug = ubase + u
            jt = ug // 64
            ic0 = 2 * (ug % 64)
            j = jt * 8 + jr
            return out_hbm.at[pl.ds(j, 1), pl.ds(8 * tr, 8),
                              pl.ds(ic0 * 128, 256)]

        def tb_src(jr, tr):
            return tbuf.at[jr, tr, :, :, pl.ds(0, 256)]

        def fire_stores(u):
            for jr in range(8):
                for tr in range(2):
                    pltpu.async_copy(tb_src(jr, tr), out_slice(u, jr, tr),
                                     osem)

        def wait_stores(u):
            for jr in range(8):
                for tr in range(2):
                    pltpu.make_async_copy(tb_src(jr, tr),
                                          out_slice(u, jr, tr), osem).wait()

        fire_load(0, buf0)
        fire_load(1, buf1)
        wait_load(0, buf0)
        transpose(buf0)
        fire_stores(0)

        def step(u, bufref, other):
            fire_load(u + 1, other)
            wait_load(u, bufref)
            wait_stores(u - 1)
            transpose(bufref)
            fire_stores(u)

        def body(q, carry):
            step(2 * q + 1, buf1, buf0)
            step(2 * q + 2, buf0, buf1)
            return carry

        lax.fori_loop(0, (per_w - 2) // 2, body, 0)

        u_last = per_w - 1
        wait_load(u_last, buf1)
        wait_stores(u_last - 1)
        transpose(buf1)
        fire_stores(u_last)
        wait_stores(u_last)

    return k(flat)


def kernel(x, W):
    Bx, H = x.shape
    V, D = W.shape
    B = Bx * H
    idx_flat = (x.reshape(Bx // 128, 128, H // 8, 8)
                .transpose(2, 0, 3, 1).reshape(B))
    tail = W[WCV_FULL * 128:].reshape(WCV_REM * D)
    Wlin = _wconv(jnp.transpose(W).reshape(2, 8, V), tail).reshape(V, D)
    out_lin = _gather_flat(idx_flat, Wlin, B, D)
    z = _relayout(out_lin.reshape(B * D), H, Bx)
    return jnp.transpose(z, (2, 0, 1))


# speedup vs baseline: 1.3445x; 1.3445x over previous
import functools

import jax
import jax.numpy as jnp
from jax import lax
from jax.experimental import pallas as pl
from jax.experimental.pallas import tpu as pltpu
from jax.experimental.pallas import tpu_sc as plsc

NUM_CORES = 2
NUM_SUBCORES = 16
NUM_WORKERS = NUM_CORES * NUM_SUBCORES
CHUNK = 2048


def _mesh():
    return plsc.VectorSubcoreMesh(
        core_axis_name="c",
        subcore_axis_name="s",
        num_cores=NUM_CORES,
        num_subcores=NUM_SUBCORES,
    )


@functools.partial(jax.jit, static_argnames=("B", "D"))
def _gather_flat(idx_flat, W, B, D):
    per_w = B // NUM_WORKERS
    n_chunks = per_w // CHUNK
    assert n_chunks % 2 == 0 and n_chunks >= 4
    n_pairs = (n_chunks - 2) // 2

    @functools.partial(
        pl.kernel,
        mesh=_mesh(),
        compiler_params=pltpu.CompilerParams(use_tc_tiling_on_sc=False),
        out_type=jax.ShapeDtypeStruct((B, D), jnp.float32),
        scratch_types=[
            pltpu.VMEM((2, CHUNK), jnp.int32),
            pltpu.VMEM((2, CHUNK, D), jnp.float32),
            pltpu.SemaphoreType.DMA,
            pltpu.SemaphoreType.DMA,
        ],
    )
    def k(idx_hbm, table_hbm, out_hbm, idx_v, rows_v, gsem, osem):
        wid = lax.axis_index("s") * NUM_CORES + lax.axis_index("c")
        base = wid * per_w

        def load_idx(g, b):
            pltpu.sync_copy(idx_hbm.at[pl.ds(base + g * CHUNK, CHUNK)],
                            idx_v.at[b])

        def fire_gather(b):
            pltpu.async_copy(table_hbm.at[idx_v.at[b]], rows_v.at[b], gsem)

        def wait_gather(b):
            pltpu.make_async_copy(table_hbm.at[idx_v.at[b]], rows_v.at[b],
                                  gsem).wait()

        def fire_store(g, b):
            pltpu.async_copy(rows_v.at[b],
                             out_hbm.at[pl.ds(base + g * CHUNK, CHUNK)], osem)

        def wait_store(g, b):
            pltpu.make_async_copy(rows_v.at[b],
                                  out_hbm.at[pl.ds(base + g * CHUNK, CHUNK)],
                                  osem).wait()

        load_idx(0, 0)
        fire_gather(0)
        load_idx(1, 1)
        wait_gather(0)
        fire_gather(1)
        fire_store(0, 0)

        def step(g, b):
            load_idx(g + 1, b ^ 1)
            wait_gather(b)
            wait_store(g - 1, b ^ 1)
            fire_gather(b ^ 1)
            fire_store(g, b)

        def body(p, carry):
            step(2 * p + 1, 1)
            step(2 * p + 2, 0)
            return carry

        lax.fori_loop(0, n_pairs, body, 0)

        g_last = n_chunks - 1
        wait_gather(1)
        wait_store(g_last - 1, 0)
        fire_store(g_last, 1)
        wait_store(g_last, 1)

    return k(idx_flat, W)


WCV_FULL = 1000000 // 128
WCV_REM = 1000000 - WCV_FULL * 128


@jax.jit
def _wconv(Wt3, tail):
    V = Wt3.shape[2]
    WB = 512
    NFULL = V // WB
    PERW = (NFULL - 1) // NUM_WORKERS

    @functools.partial(
        pl.kernel,
        mesh=_mesh(),
        compiler_params=pltpu.CompilerParams(
            use_tc_tiling_on_sc=True, needs_layout_passes=False),
        out_type=jax.ShapeDtypeStruct((V * 16,), jnp.float32),
        scratch_types=[
            pltpu.VMEM((2, 1, 8, WB), jnp.float32),
            pltpu.VMEM((2, 1, 8, WB), jnp.float32),
            pltpu.VMEM((WB * 16,), jnp.float32),
            pltpu.VMEM((WB * 16,), jnp.float32),
            pltpu.VMEM((256,), jnp.float32),
            pltpu.SemaphoreType.DMA,
            pltpu.SemaphoreType.DMA,
        ],
    )
    def k(wt_hbm, tail_hbm, out_hbm, wbufA, wbufB, obufA, obufB, sbuf,
          lsem, osem):
        wid = lax.axis_index("s") * NUM_CORES + lax.axis_index("c")
        kstart = wid * PERW
        iota = lax.iota(jnp.int32, 16)
        skew_st = [(iota + d) % 16 + d * 16 for d in range(16)]
        skew_ld = [iota * 16 + (iota + vlk) % 16 for vlk in range(16)]

        def fire_load(blk, wb):
            for tr in range(2):
                pltpu.async_copy(
                    wt_hbm.at[pl.ds(tr, 1), :,
                              pl.ds((kstart + blk) * WB, WB)],
                    wb.at[tr], lsem)

        def wait_load(blk, wb):
            for tr in range(2):
                pltpu.make_async_copy(
                    wt_hbm.at[pl.ds(tr, 1), :,
                              pl.ds((kstart + blk) * WB, WB)],
                    wb.at[tr], lsem).wait()

        def transpose(wb, ob):
            def v_body(vl16, carry):
                for tr in range(2):
                    for rr in range(8):
                        d = tr * 8 + rr
                        v = wb[tr, 0, rr, pl.ds(vl16 * 16, 16)]
                        plsc.store_scatter(sbuf, [skew_st[d]], v)
                for vlk in range(16):
                    v2 = plsc.load_gather(sbuf, [skew_ld[vlk]])
                    ob[pl.ds((vl16 * 16 + vlk) * 16, 16)] = v2
                return carry
            lax.fori_loop(0, WB // 16, v_body, 0)

        def fire_store(blk, ob):
            pltpu.async_copy(
                ob, out_hbm.at[pl.ds((kstart + blk) * WB * 16, WB * 16)],
                osem)

        def wait_store(blk, ob):
            pltpu.make_async_copy(
                ob, out_hbm.at[pl.ds((kstart + blk) * WB * 16, WB * 16)],
                osem).wait()

        fire_load(0, wbufA)
        fire_load(1, wbufB)
        wait_load(0, wbufA)
        transpose(wbufA, obufA)
        fire_store(0, obufA)
        fire_load(2, wbufA)
        wait_load(1, wbufB)
        transpose(wbufB, obufB)
        fire_store(1, obufB)

        def step(u, wb, ob, other_wb):
            fire_load(u + 1, other_wb)
            wait_load(u, wb)
            wait_store(u - 2, ob)
            transpose(wb, ob)
            fire_store(u, ob)

        def body(q, carry):
            step(2 * q + 2, wbufA, obufA, wbufB)
            step(2 * q + 3, wbufB, obufB, wbufA)
            return carry

        lax.fori_loop(0, (PERW - 3) // 2, body, 0)

        u_last = PERW - 1
        wait_load(u_last, wbufA)
        wait_store(u_last - 2, obufA)
        transpose(wbufA, obufA)
        fire_store(u_last, obufA)
        wait_store(u_last - 1, obufB)
        wait_store(u_last, obufA)

        @pl.when(wid == NUM_WORKERS - 1)
        def _extra():
            xblk = (NFULL - 1) - kstart
            for tr in range(2):
                pltpu.sync_copy(
                    wt_hbm.at[pl.ds(tr, 1), :,
                              pl.ds((kstart + xblk) * WB, WB)],
                    wbufA.at[tr])
            transpose(wbufA, obufA)
            pltpu.sync_copy(
                obufA,
                out_hbm.at[pl.ds((kstart + xblk) * WB * 16, WB * 16)])
            pltpu.sync_copy(tail_hbm, obufA.at[pl.ds(0, WCV_REM * 16)])
            pltpu.sync_copy(obufA.at[pl.ds(0, WCV_REM * 16)],
                            out_hbm.at[pl.ds(WCV_FULL * 2048, WCV_REM * 16)])

    return k(Wt3, tail)


UNITS = 1600
UNIT_ROWS = 2048
UNIT_ELEMS = UNIT_ROWS * 16


@functools.partial(jax.jit, static_argnames=("HIST", "BATCH"))
def _relayout(flat, HIST, BATCH):
    per_w = UNITS // NUM_WORKERS

    @functools.partial(
        pl.kernel,
        mesh=_mesh(),
        compiler_params=pltpu.CompilerParams(
            use_tc_tiling_on_sc=True, needs_layout_passes=False),
        out_type=jax.ShapeDtypeStruct((HIST, 16, BATCH), jnp.float32),
        scratch_types=[
            pltpu.VMEM((UNIT_ELEMS,), jnp.float32),
            pltpu.VMEM((UNIT_ELEMS,), jnp.float32),
            pltpu.VMEM((8, 2, 1, 8, 256), jnp.float32),
            pltpu.VMEM((256,), jnp.float32),
            pltpu.VMEM((256,), jnp.float32),
            pltpu.SemaphoreType.DMA,
            pltpu.SemaphoreType.DMA,
        ],
    )
    def k(in_hbm, out_hbm, buf0, buf1, tbuf, sbufA, sbufB, lsem, osem):
        wid = lax.axis_index("s") * NUM_CORES + lax.axis_index("c")
        ubase = wid * per_w
        iota = lax.iota(jnp.int32, 16)
        skew_st = [(iota + k) % 16 + k * 16 for k in range(16)]
        skew_ld = [iota * 16 + (iota + d) % 16 for d in range(16)]

        def fire_load(u, bufref):
            pltpu.async_copy(
                in_hbm.at[pl.ds((ubase + u) * UNIT_ELEMS, UNIT_ELEMS)],
                bufref, lsem)

        def wait_load(u, bufref):
            pltpu.make_async_copy(
                in_hbm.at[pl.ds((ubase + u) * UNIT_ELEMS, UNIT_ELEMS)],
                bufref, lsem).wait()

        def transpose(bufref):
            def rb_params(rb):
                jr = rb // 16
                icl = (rb % 16) // 8
                ii16 = rb % 8
                r0 = (icl * 8 + jr) * 128 + ii16 * 16
                col = icl * 128 + ii16 * 16
                return jr, r0, col

            def p_body(p, carry):
                for half, sb in ((0, sbufA), (1, sbufB)):
                    _, r0, _ = rb_params(2 * p + half)
                    for kk in range(16):
                        v = bufref[pl.ds((r0 + kk) * 16, 16)]
                        plsc.store_scatter(sb, [skew_st[kk]], v)
                for half, sb in ((0, sbufA), (1, sbufB)):
                    jr, _, col = rb_params(2 * p + half)
                    for d in range(16):
                        v2 = plsc.load_gather(sb, [skew_ld[d]])
                        tbuf[jr, d // 8, 0, d % 8, pl.ds(col, 16)] = v2
                return carry
            lax.fori_loop(0, 64, p_body, 0)

        def out_slice(u, jr, tr):
            ug = ubase + u
            jt = ug // 64
            ic0 = 2 * (ug % 64)
            j = jt * 8 + jr
            return out_hbm.at[pl.ds(j, 1), pl.ds(8 * tr, 8),
                              pl.ds(ic0 * 128, 256)]

        def fire_stores(u):
            for jr in range(8):
                for tr in range(2):
                    pltpu.async_copy(tbuf.at[jr, tr], out_slice(u, jr, tr),
                                     osem)

        def wait_stores(u):
            for jr in range(8):
                for tr in range(2):
                    pltpu.make_async_copy(tbuf.at[jr, tr],
                                          out_slice(u, jr, tr), osem).wait()

        fire_load(0, buf0)
        fire_load(1, buf1)
        wait_load(0, buf0)
        transpose(buf0)
        fire_stores(0)

        def step(u, bufref, other):
            fire_load(u + 1, other)
            wait_load(u, bufref)
            wait_stores(u - 1)
            transpose(bufref)
            fire_stores(u)

        def body(q, carry):
            step(2 * q + 1, buf1, buf0)
            step(2 * q + 2, buf0, buf1)
            return carry

        lax.fori_loop(0, (per_w - 2) // 2, body, 0)

        u_last = per_w - 1
        wait_load(u_last, buf1)
        wait_stores(u_last - 1)
        transpose(buf1)
        fire_stores(u_last)
        wait_stores(u_last)

    return k(flat)


def kernel(x, W):
    Bx, H = x.shape
    V, D = W.shape
    B = Bx * H
    idx_flat = (x.reshape(Bx // 128, 128, H // 8, 8)
                .transpose(2, 0, 3, 1).reshape(B))
    tail = W[WCV_FULL * 128:].reshape(WCV_REM * D)
    Wlin = _wconv(jnp.transpose(W).reshape(2, 8, V), tail).reshape(V, D)
    out_lin = _gather_flat(idx_flat, Wlin, B, D)
    z = _relayout(out_lin.reshape(B * D), H, Bx)
    return jnp.transpose(z, (2, 0, 1))
